# baseline (speedup 1.0000x reference)
import functools

import jax
import jax.numpy as jnp
import numpy as np
from jax import lax
from jax.experimental import pallas as pl
from jax.experimental.pallas import tpu as pltpu
from jax.experimental.pallas import tpu_sc as plsc

S = 5000
T = 5000
N = S + T
A = 16
F_IN = 8
F_OUT = 4
C_IN = 128
C_OUT = 32
H = 4
D = F_OUT * C_OUT * H

NC = 2
NS = 16
NW = NC * NS
NPW = 320
NTOT = NW * NPW
G = 4
NG = NPW // G
IDXB = G * A

_F32 = jnp.float32


def _proj_body0(x_ref, wc_ref, bc_ref, o_ref):
    y = lax.dot_general(x_ref[...], wc_ref[...], (((1,), (1,)), ((), ())),
                        preferred_element_type=_F32)
    o_ref[...] = y + bc_ref[...][None, :]


def _proj_body1(x_ref, wc_ref, bc_ref, tbl_ref, o_ref):
    del tbl_ref
    _proj_body0(x_ref, wc_ref, bc_ref, o_ref)


def _proj_side(x2, wc, bc, table_init, row_off, blk):
    grid = (S // blk,)
    specs = [
        pl.BlockSpec((blk, F_IN * C_IN), lambda i: (i, 0)),
        pl.BlockSpec((D, F_IN * C_IN), lambda i: (0, 0)),
        pl.BlockSpec((D,), lambda i: (0,)),
    ]
    out_spec = pl.BlockSpec((blk, D), lambda i, _o=row_off // blk: (i + _o, 0))
    out_shape = jax.ShapeDtypeStruct((N, D), _F32)
    if table_init is None:
        return pl.pallas_call(
            _proj_body0, grid=grid, in_specs=specs,
            out_specs=out_spec, out_shape=out_shape,
        )(x2, wc, bc)
    return pl.pallas_call(
        _proj_body1, grid=grid,
        in_specs=specs + [pl.BlockSpec(memory_space=pl.ANY)],
        out_specs=out_spec, out_shape=out_shape,
        input_output_aliases={3: 0},
    )(x2, wc, bc, table_init)


def _att_body(pts_ref, nei_ref, nb_ref, w1_ref, b1_ref, w2_ref, b2_ref,
              *out_refs):
    d = None
    for i in range(3):
        diff = pts_ref[i, :][:, None] - nei_ref[i, :, :]
        d = diff * diff if d is None else d + diff * diff
    hs = [jnp.maximum(d * w1_ref[h, 0] + b1_ref[h], 0.0) for h in range(H)]
    ats = []
    for h in range(H):
        acc = None
        for k in range(H):
            t = w2_ref[h, k] * hs[k]
            acc = t if acc is None else acc + t
        ats.append(acc + b2_ref[h])
    m = ats[0]
    for h in range(1, H):
        m = jnp.maximum(m, ats[h])
    mx = jnp.max(m, axis=1, keepdims=True)
    es = [jnp.exp(a - mx) for a in ats]
    tot = [jnp.sum(e, axis=1, keepdims=True) for e in es]
    nb = nb_ref[...]
    valid = nb < N
    for h in range(H):
        sm = es[h] / tot[h]
        out_refs[h][...] = jnp.where(valid, sm, 0.0)
    out_refs[H][...] = jnp.where(valid, nb, 0)


def _att_side(pts_t, nei_t, nb, w1, b1, w2, b2):
    smem = functools.partial(pl.BlockSpec, memory_space=pltpu.SMEM)
    outs = pl.pallas_call(
        _att_body,
        in_specs=[
            pl.BlockSpec((3, S), lambda: (0, 0)),
            pl.BlockSpec((3, S, A), lambda: (0, 0, 0)),
            pl.BlockSpec((S, A), lambda: (0, 0)),
            smem(), smem(), smem(), smem(),
        ],
        out_specs=[pl.BlockSpec((S, A), lambda: (0, 0))] * (H + 1),
        out_shape=[jax.ShapeDtypeStruct((S, A), _F32)] * H
        + [jax.ShapeDtypeStruct((S, A), jnp.int32)],
    )(pts_t, nei_t, nb, w1, b1, w2, b2)
    return outs[:H], outs[H]


def _sc_agg_body(table_hbm, idx_hbm, a_hbm, out_hbm,
                 idx_v, rb0, rb1, rb2, wb0, wb1, wb2, ob0, ob1, ob2,
                 gs0, gs1, gs2, ws0, ws1, ws2, os0, os1, os2):
    wid = lax.axis_index("s") * NC + lax.axis_index("c")
    base_row = wid * NG
    node_base = wid * NPW

    rbufs = (rb0, rb1, rb2)
    wbufs = (wb0, wb1, wb2)
    obufs = (ob0, ob1, ob2)
    gsems = (gs0, gs1, gs2)
    wsems = (ws0, ws1, ws2)
    osems = (os0, os1, os2)

    pltpu.sync_copy(idx_hbm.at[pl.ds(base_row, NG)], idx_v)

    def issue(g, slot):
        pltpu.async_copy(table_hbm.at[idx_v.at[g]], rbufs[slot], gsems[slot])
        pltpu.async_copy(a_hbm.at[base_row + g], wbufs[slot], wsems[slot])

    issue(0, 0)
    issue(1, 1)

    def step(g, slot):
        @pl.when(g + 2 < NG)
        def _():
            issue(g + 2, (slot + 2) % 3)

        rb, wb, ob = rbufs[slot], wbufs[slot], obufs[slot]
        pltpu.make_async_copy(table_hbm.at[idx_v.at[g]], rb,
                              gsems[slot]).wait()
        pltpu.make_async_copy(a_hbm.at[base_row + g], wb, wsems[slot]).wait()

        @pl.when(g >= 3)
        def _():
            pltpu.make_async_copy(
                ob, out_hbm.at[pl.ds(node_base, G)], osems[slot]).wait()

        for n in range(G):
            def a_body(u, accs, _n=n):
                q = _n * A + 2 * u
                wv0 = wb[pl.ds(q * 16, 16)]
                wv1 = wb[pl.ds(q * 16 + 16, 16)]
                return tuple(
                    accs[k] + wv0 * rb[q, pl.ds(k * 16, 16)]
                    + wv1 * rb[q + 1, pl.ds(k * 16, 16)]
                    for k in range(D // 16))

            accs = lax.fori_loop(
                0, A // 2, a_body,
                tuple(jnp.zeros((16,), _F32) for _ in range(D // 16)))
            for k in range(D // 16):
                ob[n, pl.ds(16 * k, 16)] = accs[k]

        pltpu.async_copy(ob, out_hbm.at[pl.ds(node_base + g * G, G)],
                         osems[slot])

    def triple(t, _):
        step(3 * t, 0)
        step(3 * t + 1, 1)
        step(3 * t + 2, 2)
        return _

    lax.fori_loop(0, (NG - 2) // 3, triple, 0)
    step(NG - 2, (NG - 2) % 3)
    step(NG - 1, (NG - 1) % 3)

    for slot in range(3):
        pltpu.make_async_copy(
            obufs[slot], out_hbm.at[pl.ds(node_base, G)], osems[slot]).wait()


_sc_agg = functools.partial(
    pl.kernel,
    out_type=jax.ShapeDtypeStruct((NTOT, D), _F32),
    mesh=plsc.VectorSubcoreMesh(core_axis_name="c", subcore_axis_name="s",
                                num_cores=NC, num_subcores=NS),
    scratch_types=[
        pltpu.VMEM((NG, IDXB), jnp.int32),
        pltpu.VMEM((IDXB, D), _F32),
        pltpu.VMEM((IDXB, D), _F32),
        pltpu.VMEM((IDXB, D), _F32),
        pltpu.VMEM((IDXB * A,), _F32),
        pltpu.VMEM((IDXB * A,), _F32),
        pltpu.VMEM((IDXB * A,), _F32),
        pltpu.VMEM((G, D), _F32),
        pltpu.VMEM((G, D), _F32),
        pltpu.VMEM((G, D), _F32),
    ] + [pltpu.SemaphoreType.DMA] * 9,
)(_sc_agg_body)


def _fin_body(x_ref, w_ref, b_ref, o_ref):
    blk = x_ref.shape[0]
    x = x_ref[...].reshape(blk * F_OUT, H * C_OUT)
    y = lax.dot_general(x, w_ref[0], (((1,), (1,)), ((), ())),
                        preferred_element_type=_F32)
    y = y + b_ref[0, 0][None, :]
    o_ref[0] = y.reshape(blk, F_OUT, C_OUT)


def _fin_all(agg, w2, b2, blk):
    half = S // blk
    return pl.pallas_call(
        _fin_body,
        grid=(2 * half,),
        in_specs=[
            pl.BlockSpec((blk, D), lambda i: (i, 0)),
            pl.BlockSpec((1, C_OUT, H * C_OUT), lambda i, _h=half: (i // _h, 0, 0)),
            pl.BlockSpec((1, 1, C_OUT), lambda i, _h=half: (i // _h, 0, 0)),
        ],
        out_specs=pl.BlockSpec((1, blk, F_OUT, C_OUT),
                               lambda i, _h=half: (i // _h, i % _h, 0, 0)),
        out_shape=jax.ShapeDtypeStruct((2, S, F_OUT, C_OUT), _F32),
    )(agg, w2, b2)


def kernel(src, trg, src_pts, trg_pts, src_nei_pts, trg_nei_pts, neighbor,
           src_len, trg_len, src_nei_mask, trg_nei_mask,
           w_src_w, w_src_b, w_trg_w, w_trg_b,
           out_src_w, out_src_b, out_trg_w, out_trg_b,
           satt_w1, satt_b1, satt_w2, satt_b2,
           tatt_w1, tatt_b1, tatt_w2, tatt_b2, proxy):
    neighbor = neighbor.astype(jnp.int32)

    def _fold(w, bias):
        w4 = w.reshape(C_OUT, H, C_IN)
        wc = jnp.einsum('hfo,chi->ochfi', proxy, w4).reshape(D, F_IN * C_IN)
        ps = jnp.sum(proxy, axis=1)
        bc = (ps.T[:, None, :] * bias.reshape(C_OUT, H)[None, :, :]).reshape(D)
        return wc, bc

    wc_s, bc_s = _fold(w_src_w, w_src_b)
    wc_t, bc_t = _fold(w_trg_w, w_trg_b)

    table = _proj_side(src.reshape(S, F_IN * C_IN), wc_s, bc_s, None, 0, 1000)
    table = _proj_side(trg.reshape(T, F_IN * C_IN), wc_t, bc_t, table, S, 1000)

    sm_s, idx_s = _att_side(src_pts.T, jnp.transpose(src_nei_pts, (2, 0, 1)),
                            neighbor[:S], satt_w1, satt_b1, satt_w2, satt_b2)
    sm_t, idx_t = _att_side(trg_pts.T, jnp.transpose(trg_nei_pts, (2, 0, 1)),
                            neighbor[S:], tatt_w1, tatt_b1, tatt_w2, tatt_b2)

    st = jnp.concatenate([
        jnp.stack(sm_s, axis=-1), jnp.stack(sm_t, axis=-1),
        jnp.zeros((NTOT - N, A, H), _F32),
    ], axis=0)
    expand = jnp.asarray(np.equal.outer(np.arange(H),
                                        np.arange(16) % H).astype(np.float32))
    aexp = jnp.einsum('nah,hl->nal', st, expand).reshape(NW * NG, IDXB * A)

    idx_all = jnp.concatenate(
        [idx_s, idx_t, jnp.zeros((NTOT - N, A), jnp.int32)],
        axis=0).reshape(NW * NG, IDXB)

    agg = _sc_agg(table, idx_all, aexp)

    outs = _fin_all(agg, jnp.stack([out_src_w, out_trg_w]),
                    jnp.stack([out_src_b, out_trg_b]).reshape(2, 1, C_OUT),
                    1000)
    return (outs[0], outs[1])


# speedup vs baseline: 1.0164x; 1.0164x over previous
import functools

import jax
import jax.numpy as jnp
import numpy as np
from jax import lax
from jax.experimental import pallas as pl
from jax.experimental.pallas import tpu as pltpu
from jax.experimental.pallas import tpu_sc as plsc

S = 5000
T = 5000
N = S + T
A = 16
F_IN = 8
F_OUT = 4
C_IN = 128
C_OUT = 32
H = 4
D = F_OUT * C_OUT * H

NC = 2
NS = 16
NW = NC * NS
NPW = 320
NTOT = NW * NPW
G = 4
NG = NPW // G
IDXB = G * A

_F32 = jnp.float32


def _proj_body0(x_ref, wc_ref, bc_ref, o_ref):
    y = lax.dot_general(x_ref[...], wc_ref[...], (((1,), (1,)), ((), ())),
                        preferred_element_type=_F32)
    o_ref[...] = y + bc_ref[...][None, :]


def _proj_body1(x_ref, wc_ref, bc_ref, tbl_ref, o_ref):
    del tbl_ref
    _proj_body0(x_ref, wc_ref, bc_ref, o_ref)


def _proj_side(x2, wc, bc, table_init, row_off, blk):
    grid = (S // blk,)
    specs = [
        pl.BlockSpec((blk, F_IN * C_IN), lambda i: (i, 0)),
        pl.BlockSpec((D, F_IN * C_IN), lambda i: (0, 0)),
        pl.BlockSpec((D,), lambda i: (0,)),
    ]
    out_spec = pl.BlockSpec((blk, D), lambda i, _o=row_off // blk: (i + _o, 0))
    out_shape = jax.ShapeDtypeStruct((N, D), _F32)
    if table_init is None:
        return pl.pallas_call(
            _proj_body0, grid=grid, in_specs=specs,
            out_specs=out_spec, out_shape=out_shape,
        )(x2, wc, bc)
    return pl.pallas_call(
        _proj_body1, grid=grid,
        in_specs=specs + [pl.BlockSpec(memory_space=pl.ANY)],
        out_specs=out_spec, out_shape=out_shape,
        input_output_aliases={3: 0},
    )(x2, wc, bc, table_init)


def _att_body(pts_ref, nei_ref, nb_ref, w1_ref, b1_ref, w2_ref, b2_ref,
              *out_refs):
    d = None
    for i in range(3):
        diff = pts_ref[i, :][:, None] - nei_ref[i, :, :]
        d = diff * diff if d is None else d + diff * diff
    hs = [jnp.maximum(d * w1_ref[h, 0] + b1_ref[h], 0.0) for h in range(H)]
    ats = []
    for h in range(H):
        acc = None
        for k in range(H):
            t = w2_ref[h, k] * hs[k]
            acc = t if acc is None else acc + t
        ats.append(acc + b2_ref[h])
    m = ats[0]
    for h in range(1, H):
        m = jnp.maximum(m, ats[h])
    mx = jnp.max(m, axis=1, keepdims=True)
    es = [jnp.exp(a - mx) for a in ats]
    tot = [jnp.sum(e, axis=1, keepdims=True) for e in es]
    nb = nb_ref[...]
    valid = nb < N
    for h in range(H):
        sm = es[h] / tot[h]
        out_refs[h][...] = jnp.where(valid, sm, 0.0)
    out_refs[H][...] = jnp.where(valid, nb, 0)


def _att_side(pts_t, nei_t, nb, w1, b1, w2, b2):
    smem = functools.partial(pl.BlockSpec, memory_space=pltpu.SMEM)
    outs = pl.pallas_call(
        _att_body,
        in_specs=[
            pl.BlockSpec((3, S), lambda: (0, 0)),
            pl.BlockSpec((3, S, A), lambda: (0, 0, 0)),
            pl.BlockSpec((S, A), lambda: (0, 0)),
            smem(), smem(), smem(), smem(),
        ],
        out_specs=[pl.BlockSpec((S, A), lambda: (0, 0))] * (H + 1),
        out_shape=[jax.ShapeDtypeStruct((S, A), _F32)] * H
        + [jax.ShapeDtypeStruct((S, A), jnp.int32)],
    )(pts_t, nei_t, nb, w1, b1, w2, b2)
    return outs[:H], outs[H]


def _sc_agg_body(table_hbm, idx_hbm, a_hbm, out_hbm,
                 idx_v, rb0, rb1, rb2, wb0, wb1, wb2, ob0, ob1, ob2,
                 gs0, gs1, gs2, ws0, ws1, ws2, os0, os1, os2):
    wid = lax.axis_index("s") * NC + lax.axis_index("c")
    base_row = wid * NG
    node_base = wid * NPW

    rbufs = (rb0, rb1, rb2)
    wbufs = (wb0, wb1, wb2)
    obufs = (ob0, ob1, ob2)
    gsems = (gs0, gs1, gs2)
    wsems = (ws0, ws1, ws2)
    osems = (os0, os1, os2)

    pltpu.sync_copy(idx_hbm.at[pl.ds(base_row, NG)], idx_v)

    def issue(g, slot):
        pltpu.async_copy(table_hbm.at[idx_v.at[g]], rbufs[slot], gsems[slot])
        pltpu.async_copy(a_hbm.at[base_row + g], wbufs[slot], wsems[slot])

    issue(0, 0)
    issue(1, 1)

    def step(g, slot):
        @pl.when(g + 2 < NG)
        def _():
            issue(g + 2, (slot + 2) % 3)

        rb, wb, ob = rbufs[slot], wbufs[slot], obufs[slot]
        pltpu.make_async_copy(table_hbm.at[idx_v.at[g]], rb,
                              gsems[slot]).wait()
        pltpu.make_async_copy(a_hbm.at[base_row + g], wb, wsems[slot]).wait()

        @pl.when(g >= 3)
        def _():
            pltpu.make_async_copy(
                ob, out_hbm.at[pl.ds(node_base, G)], osems[slot]).wait()

        for n in range(G):
            def a_body(u, accs, _n=n):
                q = _n * A + 2 * u
                wv0 = wb[pl.ds(q * 16, 16)]
                wv1 = wb[pl.ds(q * 16 + 16, 16)]
                return tuple(
                    accs[k] + wv0 * rb[q, pl.ds(k * 16, 16)]
                    + wv1 * rb[q + 1, pl.ds(k * 16, 16)]
                    for k in range(D // 16))

            accs = lax.fori_loop(
                0, A // 2, a_body,
                tuple(jnp.zeros((16,), _F32) for _ in range(D // 16)))
            for k in range(D // 16):
                ob[n, pl.ds(16 * k, 16)] = accs[k]

        pltpu.async_copy(ob, out_hbm.at[pl.ds(node_base + g * G, G)],
                         osems[slot])

    def triple(t, _):
        step(3 * t, 0)
        step(3 * t + 1, 1)
        step(3 * t + 2, 2)
        return _

    lax.fori_loop(0, (NG - 2) // 3, triple, 0)
    step(NG - 2, (NG - 2) % 3)
    step(NG - 1, (NG - 1) % 3)

    for slot in range(3):
        pltpu.make_async_copy(
            obufs[slot], out_hbm.at[pl.ds(node_base, G)], osems[slot]).wait()


_sc_agg = functools.partial(
    pl.kernel,
    out_type=jax.ShapeDtypeStruct((NTOT, D), _F32),
    mesh=plsc.VectorSubcoreMesh(core_axis_name="c", subcore_axis_name="s",
                                num_cores=NC, num_subcores=NS),
    scratch_types=[
        pltpu.VMEM((NG, IDXB), jnp.int32),
        pltpu.VMEM((IDXB, D), _F32),
        pltpu.VMEM((IDXB, D), _F32),
        pltpu.VMEM((IDXB, D), _F32),
        pltpu.VMEM((IDXB * A,), _F32),
        pltpu.VMEM((IDXB * A,), _F32),
        pltpu.VMEM((IDXB * A,), _F32),
        pltpu.VMEM((G, D), _F32),
        pltpu.VMEM((G, D), _F32),
        pltpu.VMEM((G, D), _F32),
    ] + [pltpu.SemaphoreType.DMA] * 9,
)(_sc_agg_body)


def _fin_body(x_ref, w_ref, b_ref, o_ref):
    blk = x_ref.shape[0]
    x = x_ref[...].reshape(blk * F_OUT, H * C_OUT)
    y = lax.dot_general(x, w_ref[...], (((1,), (1,)), ((), ())),
                        preferred_element_type=_F32)
    y = y + b_ref[...][None, :]
    o_ref[...] = y.reshape(blk, F_OUT, C_OUT)


def _fin_side(agg, w, bias, row_off, blk):
    grid = (S // blk,)
    return pl.pallas_call(
        _fin_body,
        grid=grid,
        in_specs=[
            pl.BlockSpec((blk, D), lambda i, _o=row_off // blk: (i + _o, 0)),
            pl.BlockSpec((C_OUT, H * C_OUT), lambda i: (0, 0)),
            pl.BlockSpec((C_OUT,), lambda i: (0,)),
        ],
        out_specs=pl.BlockSpec((blk, F_OUT, C_OUT), lambda i: (i, 0, 0)),
        out_shape=jax.ShapeDtypeStruct((S, F_OUT, C_OUT), _F32),
    )(agg, w, bias)


def kernel(src, trg, src_pts, trg_pts, src_nei_pts, trg_nei_pts, neighbor,
           src_len, trg_len, src_nei_mask, trg_nei_mask,
           w_src_w, w_src_b, w_trg_w, w_trg_b,
           out_src_w, out_src_b, out_trg_w, out_trg_b,
           satt_w1, satt_b1, satt_w2, satt_b2,
           tatt_w1, tatt_b1, tatt_w2, tatt_b2, proxy):
    neighbor = neighbor.astype(jnp.int32)

    def _fold(w, bias):
        w4 = w.reshape(C_OUT, H, C_IN)
        wc = jnp.einsum('hfo,chi->ochfi', proxy, w4).reshape(D, F_IN * C_IN)
        ps = jnp.sum(proxy, axis=1)
        bc = (ps.T[:, None, :] * bias.reshape(C_OUT, H)[None, :, :]).reshape(D)
        return wc, bc

    wc_s, bc_s = _fold(w_src_w, w_src_b)
    wc_t, bc_t = _fold(w_trg_w, w_trg_b)

    table = _proj_side(src.reshape(S, F_IN * C_IN), wc_s, bc_s, None, 0, 1000)
    table = _proj_side(trg.reshape(T, F_IN * C_IN), wc_t, bc_t, table, S, 1000)

    sm_s, idx_s = _att_side(src_pts.T, jnp.transpose(src_nei_pts, (2, 0, 1)),
                            neighbor[:S], satt_w1, satt_b1, satt_w2, satt_b2)
    sm_t, idx_t = _att_side(trg_pts.T, jnp.transpose(trg_nei_pts, (2, 0, 1)),
                            neighbor[S:], tatt_w1, tatt_b1, tatt_w2, tatt_b2)

    st = jnp.concatenate([
        jnp.stack(sm_s, axis=-1), jnp.stack(sm_t, axis=-1),
        jnp.zeros((NTOT - N, A, H), _F32),
    ], axis=0)
    expand = jnp.asarray(np.equal.outer(np.arange(H),
                                        np.arange(16) % H).astype(np.float32))
    aexp = jnp.einsum('nah,hl->nal', st, expand).reshape(NW * NG, IDXB * A)

    idx_all = jnp.concatenate(
        [idx_s, idx_t, jnp.zeros((NTOT - N, A), jnp.int32)],
        axis=0).reshape(NW * NG, IDXB)

    agg = _sc_agg(table, idx_all, aexp)

    s_out = _fin_side(agg, out_src_w, out_src_b, 0, 1000)
    t_out = _fin_side(agg, out_trg_w, out_trg_b, S, 1000)
    return (s_out, t_out)
